# Initial kernel scaffold; baseline (speedup 1.0000x reference)
import jax
import jax.numpy as jnp
from jax.experimental import pallas as pl


def kernel(x, segment_info, table, seg_table, pos):
    raise NotImplementedError("write your pallas kernel here")


# speedup vs baseline: 1.1313x; 1.1313x over previous
import functools

import jax
import jax.numpy as jnp
from jax import lax
from jax.experimental import pallas as pl
from jax.experimental.pallas import tpu as pltpu
from jax.experimental.pallas import tpu_sc as plsc

B, L, V, D = 1024, 200, 1000000, 64
N = B * L
NW = 32
PER_W = N // NW
C = 128
NCH = PER_W // C
LANES = 16


def _comb_body(seg_ref, pos_ref, out_ref):
    out_ref[...] = seg_ref[...][:, None, :] + pos_ref[...][None, :, :]


def _build_comb(seg_table, pos200):
    return pl.pallas_call(
        _comb_body,
        out_shape=jax.ShapeDtypeStruct((3, L, D), jnp.float32),
    )(seg_table, pos200)


_MESH = plsc.VectorSubcoreMesh(core_axis_name="c", subcore_axis_name="s")


@functools.partial(
    pl.kernel,
    mesh=_MESH,
    compiler_params=pltpu.CompilerParams(use_tc_tiling_on_sc=False),
    out_type=jax.ShapeDtypeStruct((N, D), jnp.float32),
    scratch_types=[
        pltpu.VMEM((C,), jnp.int32),
        pltpu.VMEM((C,), jnp.int32),
        pltpu.VMEM((C,), jnp.int32),
        pltpu.VMEM((C, D), jnp.float32),
        pltpu.VMEM((C, D), jnp.float32),
        pltpu.SemaphoreType.DMA,
        pltpu.SemaphoreType.DMA,
    ],
)
def _sc_embed(x_hbm, sg_hbm, table_hbm, comb_hbm, out_hbm,
              idx_v, sidx_v, cidx_v, rows_v, crows_v, sem, sem2):
    cid = lax.axis_index("c")
    sid = lax.axis_index("s")
    wid = sid * 2 + cid
    base = wid * PER_W

    def chunk_body(ci, carry):
        g = base + ci * C
        pltpu.sync_copy(x_hbm.at[pl.ds(g, C)], idx_v)
        pltpu.sync_copy(sg_hbm.at[pl.ds(g, C)], sidx_v)
        gather = pltpu.async_copy(table_hbm.at[idx_v], rows_v, sem)

        def lane_body(j, carry2):
            lane = lax.iota(jnp.int32, 16)
            tok = g + j * LANES + lane
            seg = sidx_v[pl.ds(j * LANES, LANES)]
            cidx_v[pl.ds(j * LANES, LANES)] = seg * L + tok % L
            return carry2

        lax.fori_loop(0, C // LANES, lane_body, 0)
        gather.wait()
        pltpu.async_copy(comb_hbm.at[cidx_v], crows_v, sem2).wait()

        def add_body(t, carry2):
            for dp in range(D // LANES):
                sl = pl.ds(dp * LANES, LANES)
                rows_v[t, sl] = rows_v[t, sl] + crows_v[t, sl]
            return carry2

        lax.fori_loop(0, C, add_body, 0)
        pltpu.sync_copy(rows_v, out_hbm.at[pl.ds(g, C)])
        return carry

    lax.fori_loop(0, NCH, chunk_body, 0)


def kernel(x, segment_info, table, seg_table, pos):
    xi = x.reshape(N).astype(jnp.int32)
    si = segment_info.reshape(N).astype(jnp.int32)
    comb = _build_comb(seg_table.astype(jnp.float32),
                       pos[:L].astype(jnp.float32)).reshape(3 * L, D)
    out = _sc_embed(xi, si, table, comb)
    return out.reshape(B, L, D)


# speedup vs baseline: 1.1624x; 1.0275x over previous
import functools

import jax
import jax.numpy as jnp
from jax import lax
from jax.experimental import pallas as pl
from jax.experimental.pallas import tpu as pltpu
from jax.experimental.pallas import tpu_sc as plsc

B, L, V, D = 1024, 200, 1000000, 64
N = B * L
NW = 32
PER_W = N // NW
C = 128
NCH = PER_W // C
LANES = 16


def _comb_body(seg_ref, pos_ref, out_ref):
    out_ref[...] = seg_ref[...][:, None, :] + pos_ref[...][None, :, :]


def _build_comb(seg_table, pos200):
    return pl.pallas_call(
        _comb_body,
        out_shape=jax.ShapeDtypeStruct((3, L, D), jnp.float32),
    )(seg_table, pos200)


_MESH = plsc.VectorSubcoreMesh(core_axis_name="c", subcore_axis_name="s")


@functools.partial(
    pl.kernel,
    mesh=_MESH,
    compiler_params=pltpu.CompilerParams(use_tc_tiling_on_sc=False),
    out_type=jax.ShapeDtypeStruct((N, D), jnp.float32),
    scratch_types=[
        pltpu.VMEM((C,), jnp.int32),
        pltpu.VMEM((C,), jnp.int32),
        pltpu.VMEM((C,), jnp.int32),
        pltpu.VMEM((C, D), jnp.float32),
        pltpu.VMEM((C, D), jnp.float32),
        pltpu.SemaphoreType.DMA,
        pltpu.SemaphoreType.DMA,
    ],
)
def _sc_embed(x_hbm, sg_hbm, table_hbm, comb_hbm, out_hbm,
              idx_v, sidx_v, cidx_v, rows_v, crows_v, sem, sem2):
    cid = lax.axis_index("c")
    sid = lax.axis_index("s")
    wid = sid * 2 + cid
    base = wid * PER_W

    def chunk_body(ci, carry):
        g = base + ci * C
        pltpu.sync_copy(x_hbm.at[pl.ds(g, C)], idx_v)
        pltpu.sync_copy(sg_hbm.at[pl.ds(g, C)], sidx_v)
        gather = pltpu.async_copy(table_hbm.at[idx_v], rows_v, sem)

        def lane_body(j, carry2):
            lane = lax.iota(jnp.int32, 16)
            tok = g + j * LANES + lane
            seg = sidx_v[pl.ds(j * LANES, LANES)]
            cidx_v[pl.ds(j * LANES, LANES)] = seg * L + tok % L
            return carry2

        lax.fori_loop(0, C // LANES, lane_body, 0)
        gather.wait()
        pltpu.async_copy(comb_hbm.at[cidx_v], rows_v, sem2, add=True).wait()
        pltpu.sync_copy(rows_v, out_hbm.at[pl.ds(g, C)])
        return carry

    lax.fori_loop(0, NCH, chunk_body, 0)


def kernel(x, segment_info, table, seg_table, pos):
    xi = x.reshape(N).astype(jnp.int32)
    si = segment_info.reshape(N).astype(jnp.int32)
    comb = _build_comb(seg_table.astype(jnp.float32),
                       pos[:L].astype(jnp.float32)).reshape(3 * L, D)
    out = _sc_embed(xi, si, table, comb)
    return out.reshape(B, L, D)


# speedup vs baseline: 1.2640x; 1.0874x over previous
import functools

import jax
import jax.numpy as jnp
from jax import lax
from jax.experimental import pallas as pl
from jax.experimental.pallas import tpu as pltpu
from jax.experimental.pallas import tpu_sc as plsc

B, L, V, D = 1024, 200, 1000000, 64
N = B * L
NW = 32
PER_W = N // NW
G = 128
SG = 5
C = G * SG
NCH = PER_W // C
LANES = 16


def _comb_body(seg_ref, pos_ref, out_ref):
    out_ref[...] = seg_ref[...][:, None, :] + pos_ref[...][None, :, :]


def _build_comb(seg_table, pos200):
    return pl.pallas_call(
        _comb_body,
        out_shape=jax.ShapeDtypeStruct((3, L, D), jnp.float32),
    )(seg_table, pos200)


_MESH = plsc.VectorSubcoreMesh(core_axis_name="c", subcore_axis_name="s")


@functools.partial(
    pl.kernel,
    mesh=_MESH,
    compiler_params=pltpu.CompilerParams(use_tc_tiling_on_sc=False),
    out_type=jax.ShapeDtypeStruct((N, D), jnp.float32),
    scratch_types=[
        pltpu.VMEM((SG, G), jnp.int32),
        pltpu.VMEM((C,), jnp.int32),
        pltpu.VMEM((SG, G), jnp.int32),
        pltpu.VMEM((C, D), jnp.float32),
        pltpu.SemaphoreType.DMA,
        pltpu.SemaphoreType.DMA,
    ],
)
def _sc_embed(x_hbm, sg_hbm, table_hbm, comb_hbm, out_hbm,
              idx_v, sidx_v, cidx_v, rows_v, sem, sem2):
    cid = lax.axis_index("c")
    sid = lax.axis_index("s")
    wid = sid * 2 + cid
    base = wid * PER_W

    def chunk_body(ci, carry):
        g = base + ci * C
        pltpu.sync_copy(x_hbm.at[pl.ds(g // G, SG)], idx_v)
        pltpu.sync_copy(sg_hbm.at[pl.ds(g, C)], sidx_v)
        gathers = []
        for j in range(SG):
            gathers.append(pltpu.async_copy(
                table_hbm.at[idx_v.at[j]],
                rows_v.at[pl.ds(j * G, G)], sem))

        def lane_body(k, carry2):
            lane = lax.iota(jnp.int32, 16)
            tok = g + k * LANES + lane
            seg = sidx_v[pl.ds(k * LANES, LANES)]
            j = k // (G // LANES)
            col = (k % (G // LANES)) * LANES
            cidx_v[j, pl.ds(col, LANES)] = seg * L + tok % L
            return carry2

        lax.fori_loop(0, C // LANES, lane_body, 0)
        for cp in gathers:
            cp.wait()
        adds = []
        for j in range(SG):
            adds.append(pltpu.async_copy(
                comb_hbm.at[cidx_v.at[j]],
                rows_v.at[pl.ds(j * G, G)], sem2, add=True))
        for cp in adds:
            cp.wait()
        pltpu.sync_copy(rows_v, out_hbm.at[pl.ds(g, C)])
        return carry

    lax.fori_loop(0, NCH, chunk_body, 0)


def kernel(x, segment_info, table, seg_table, pos):
    xi = x.reshape(N // G, G).astype(jnp.int32)
    si = segment_info.reshape(N).astype(jnp.int32)
    comb = _build_comb(seg_table.astype(jnp.float32),
                       pos[:L].astype(jnp.float32)).reshape(3 * L, D)
    out = _sc_embed(xi, si, table, comb)
    return out.reshape(B, L, D)


# speedup vs baseline: 1.2694x; 1.0043x over previous
import functools

import jax
import jax.numpy as jnp
from jax import lax
from jax.experimental import pallas as pl
from jax.experimental.pallas import tpu as pltpu
from jax.experimental.pallas import tpu_sc as plsc

B, L, V, D = 1024, 200, 1000000, 64
NW = 32
ROWS_W = B // NW
RC = 4
NCH = ROWS_W // RC
HALVES = ((0, 104), (104, 96))
LANES = 16


def _comb_body(seg_ref, pos_ref, out_ref):
    out_ref[...] = seg_ref[...][:, None, :] + pos_ref[...][None, :, :]


def _build_comb(seg_table, pos200):
    return pl.pallas_call(
        _comb_body,
        out_shape=jax.ShapeDtypeStruct((3, L, D), jnp.float32),
    )(seg_table, pos200)


_MESH = plsc.VectorSubcoreMesh(core_axis_name="c", subcore_axis_name="s")


@functools.partial(
    pl.kernel,
    mesh=_MESH,
    compiler_params=pltpu.CompilerParams(use_tc_tiling_on_sc=False),
    out_type=jax.ShapeDtypeStruct((B, L, D), jnp.float32),
    scratch_types=[
        pltpu.VMEM((RC, L), jnp.int32),
        pltpu.VMEM((RC, L), jnp.int32),
        pltpu.VMEM((RC, L), jnp.int32),
        pltpu.VMEM((RC, L, D), jnp.float32),
        pltpu.SemaphoreType.DMA,
        pltpu.SemaphoreType.DMA,
    ],
)
def _sc_embed(x_hbm, sg_hbm, table_hbm, comb_hbm, out_hbm,
              idx_v, sidx_v, cidx_v, rows_v, sem, sem2):
    cid = lax.axis_index("c")
    sid = lax.axis_index("s")
    wid = sid * 2 + cid
    row_base = wid * ROWS_W

    col_starts = [16 * j for j in range(L // LANES)] + [L - LANES]

    def chunk_body(ci, carry):
        b0 = row_base + ci * RC
        pltpu.sync_copy(x_hbm.at[pl.ds(b0, RC)], idx_v)
        pltpu.sync_copy(sg_hbm.at[pl.ds(b0, RC)], sidx_v)
        gathers = []
        for r in range(RC):
            for c0, w in HALVES:
                gathers.append(pltpu.async_copy(
                    table_hbm.at[idx_v.at[r, pl.ds(c0, w)]],
                    rows_v.at[r, pl.ds(c0, w)], sem))
        lane = lax.iota(jnp.int32, LANES)
        for r in range(RC):
            for c0 in col_starts:
                seg = sidx_v[r, pl.ds(c0, LANES)]
                cidx_v[r, pl.ds(c0, LANES)] = seg * L + (c0 + lane)
        for cp in gathers:
            cp.wait()
        adds = []
        for r in range(RC):
            for c0, w in HALVES:
                adds.append(pltpu.async_copy(
                    comb_hbm.at[cidx_v.at[r, pl.ds(c0, w)]],
                    rows_v.at[r, pl.ds(c0, w)], sem2, add=True))
        for cp in adds:
            cp.wait()
        pltpu.sync_copy(rows_v, out_hbm.at[pl.ds(b0, RC)])
        return carry

    lax.fori_loop(0, NCH, chunk_body, 0)


def kernel(x, segment_info, table, seg_table, pos):
    xi = x.astype(jnp.int32)
    si = segment_info.astype(jnp.int32)
    comb = _build_comb(seg_table.astype(jnp.float32),
                       pos[:L].astype(jnp.float32)).reshape(3 * L, D)
    return _sc_embed(xi, si, table, comb)
